# baseline (speedup 1.0000x reference)
import functools

import jax
import jax.numpy as jnp
from jax import lax
from jax.experimental import pallas as pl
from jax.experimental.pallas import tpu as pltpu
from jax.experimental.pallas import tpu_sc as plsc

B = 4096
D = 1024
BINS = 64
NC = 2
NS = 16
L = 16
NW = NC * NS
ROWS_PER_W = B // NW
CH = 16
NCHUNK = ROWS_PER_W // CH
VPR = D // L
U = 8
KH = 8

_mesh = plsc.VectorSubcoreMesh(core_axis_name="c", subcore_axis_name="s")


@functools.partial(
    pl.kernel,
    out_type=jax.ShapeDtypeStruct((NW, L), jnp.float32),
    mesh=_mesh,
    compiler_params=pltpu.CompilerParams(needs_layout_passes=False),
    scratch_types=[
        pltpu.VMEM((CH, D), jnp.float32),
        pltpu.VMEM((CH, D), jnp.float32),
        pltpu.VMEM((BINS,), jnp.int32),
        pltpu.VMEM((L,), jnp.float32),
        pltpu.SemaphoreType.DMA,
        pltpu.SemaphoreType.DMA,
    ],
)
def _chi2_kernel(emb_hbm, out_hbm, buf0, buf1, hist, accv, sem0, sem1):
    cid = lax.axis_index("c")
    sid = lax.axis_index("s")
    wid = sid * NC + cid
    base = wid * ROWS_PER_W
    bufs = (buf0, buf1)
    sems = (sem0, sem1)

    handles = [None, None]
    handles[0] = pltpu.async_copy(emb_hbm.at[pl.ds(base, CH)], buf0, sem0)

    zeros = jnp.zeros((L,), jnp.float32)
    izeros = jnp.zeros((L,), jnp.int32)
    iones = jnp.ones((L,), jnp.int32)
    ones = jnp.ones((L,), jnp.float32)
    expected = jnp.full((L,), B / BINS, jnp.float32)
    ksplat = [jnp.full((L,), k, jnp.int32) for k in range(KH)]
    acc = zeros

    for hb in range(BINS // L):
        hist[pl.ds(hb * L, L)] = izeros

    for c in range(NCHUNK):
        buf = bufs[c % 2]
        if c + 1 < NCHUNK:
            handles[(c + 1) % 2] = pltpu.async_copy(
                emb_hbm.at[pl.ds(base + (c + 1) * CH, CH)],
                bufs[(c + 1) % 2], sems[(c + 1) % 2])
        handles[c % 2].wait()

        def row_body(r, acc):
            carry0 = (tuple(jnp.full((L,), jnp.inf, jnp.float32)
                            for _ in range(U)),
                      tuple(jnp.full((L,), -jnp.inf, jnp.float32)
                            for _ in range(U)))

            @plsc.parallel_loop(0, VPR // U, carry=carry0)
            def mnmx(ii, carry):
                mns, mxs = carry
                i0 = ii * U
                new_mns = []
                new_mxs = []
                for u in range(U):
                    v = buf[r, pl.ds((i0 + u) * L, L)]
                    new_mns.append(jnp.minimum(mns[u], v))
                    new_mxs.append(jnp.maximum(mxs[u], v))
                return tuple(new_mns), tuple(new_mxs)

            mns, mxs = mnmx
            mn_v, mx_v = mns[0], mxs[0]
            for u in range(1, U):
                mn_v = jnp.minimum(mn_v, mns[u])
                mx_v = jnp.maximum(mx_v, mxs[u])
            mn = jnp.min(mn_v)
            mx = jnp.max(mx_v)
            delta = (mx - mn) * (1.0 / BINS)
            delta_v = jnp.broadcast_to(delta, (L,))
            inv = jnp.where(delta_v > 0, ones / delta_v, zeros)

            @plsc.parallel_loop(0, VPR, unroll=2 * U)
            def binb(i):
                v = buf[r, pl.ds(i * L, L)]
                t = (v - mn) * inv
                bidx = jnp.minimum(t.astype(jnp.int32), BINS - 1)
                plsc.addupdate_scatter(hist, [bidx], iones)

            for hb in range(BINS // L):
                h = hist[pl.ds(hb * L, L)]
                hist[pl.ds(hb * L, L)] = izeros
                dv = h.astype(jnp.float32) - expected
                acc = acc + dv * dv
            return acc

        acc = lax.fori_loop(0, CH, row_body, acc)

    accv[...] = acc
    pltpu.sync_copy(accv, out_hbm.at[wid])


def kernel(embeddings):
    partials = _chi2_kernel(embeddings)
    return jnp.sum(partials) * (1.0 / ((B / BINS + 1e-8) * B))


# speedup vs baseline: 1.0065x; 1.0065x over previous
import functools

import jax
import jax.numpy as jnp
from jax import lax
from jax.experimental import pallas as pl
from jax.experimental.pallas import tpu as pltpu
from jax.experimental.pallas import tpu_sc as plsc

B = 4096
D = 1024
BINS = 64
NC = 2
NS = 16
L = 16
NW = NC * NS
ROWS_PER_W = B // NW
CH = 16
NCHUNK = ROWS_PER_W // CH
VPR = D // L
U = 8

_mesh = plsc.VectorSubcoreMesh(core_axis_name="c", subcore_axis_name="s")


@functools.partial(
    pl.kernel,
    out_type=jax.ShapeDtypeStruct((NW, L), jnp.float32),
    mesh=_mesh,
    compiler_params=pltpu.CompilerParams(needs_layout_passes=False),
    scratch_types=[
        pltpu.VMEM((CH, D), jnp.float32),
        pltpu.VMEM((CH, D), jnp.float32),
        pltpu.VMEM((BINS,), jnp.int32),
        pltpu.VMEM((L,), jnp.float32),
        pltpu.SemaphoreType.DMA,
        pltpu.SemaphoreType.DMA,
    ],
)
def _chi2_kernel(emb_hbm, out_hbm, buf0, buf1, hist, accv, sem0, sem1):
    cid = lax.axis_index("c")
    sid = lax.axis_index("s")
    wid = sid * NC + cid
    base = wid * ROWS_PER_W
    bufs = (buf0, buf1)
    sems = (sem0, sem1)

    handles = [None, None]
    handles[0] = pltpu.async_copy(emb_hbm.at[pl.ds(base, CH)], buf0, sem0)

    zeros = jnp.zeros((L,), jnp.float32)
    izeros = jnp.zeros((L,), jnp.int32)
    iones = jnp.ones((L,), jnp.int32)
    ones = jnp.ones((L,), jnp.float32)
    expected = jnp.full((L,), B / BINS, jnp.float32)
    acc = zeros

    for hb in range(BINS // L):
        hist[pl.ds(hb * L, L)] = izeros

    for c in range(NCHUNK):
        buf = bufs[c % 2]
        if c + 1 < NCHUNK:
            handles[(c + 1) % 2] = pltpu.async_copy(
                emb_hbm.at[pl.ds(base + (c + 1) * CH, CH)],
                bufs[(c + 1) % 2], sems[(c + 1) % 2])
        handles[c % 2].wait()

        def row_body(r, acc):
            carry0 = (tuple(jnp.full((L,), jnp.inf, jnp.float32)
                            for _ in range(U)),
                      tuple(jnp.full((L,), -jnp.inf, jnp.float32)
                            for _ in range(U)))

            @plsc.parallel_loop(0, VPR // U, carry=carry0)
            def mnmx(ii, carry):
                mns, mxs = carry
                i0 = ii * U
                new_mns = []
                new_mxs = []
                for u in range(U):
                    v = buf[r, pl.ds((i0 + u) * L, L)]
                    new_mns.append(jnp.minimum(mns[u], v))
                    new_mxs.append(jnp.maximum(mxs[u], v))
                return tuple(new_mns), tuple(new_mxs)

            mns, mxs = mnmx
            mn_v, mx_v = mns[0], mxs[0]
            for u in range(1, U):
                mn_v = jnp.minimum(mn_v, mns[u])
                mx_v = jnp.maximum(mx_v, mxs[u])
            mn = jnp.min(mn_v)
            mx = jnp.max(mx_v)
            delta = (mx - mn) * (1.0 / BINS)
            delta_v = jnp.broadcast_to(delta, (L,))
            inv = jnp.where(delta_v > 0, ones / delta_v, zeros)

            @plsc.parallel_loop(0, VPR, unroll=U)
            def binb(i):
                v = buf[r, pl.ds(i * L, L)]
                t = (v - mn) * inv
                bidx = jnp.minimum(t.astype(jnp.int32), BINS - 1)
                plsc.addupdate_scatter(hist, [bidx], iones)

            for hb in range(BINS // L):
                h = hist[pl.ds(hb * L, L)]
                hist[pl.ds(hb * L, L)] = izeros
                dv = h.astype(jnp.float32) - expected
                acc = acc + dv * dv
            return acc

        acc = lax.fori_loop(0, CH, row_body, acc)

    accv[...] = acc
    pltpu.sync_copy(accv, out_hbm.at[wid])


def kernel(embeddings):
    partials = _chi2_kernel(embeddings)
    return jnp.sum(partials) * (1.0 / ((B / BINS + 1e-8) * B))
